# baseline (speedup 1.0000x reference)
import functools

import jax
import jax.numpy as jnp
from jax import lax
from jax.experimental import pallas as pl
from jax.experimental.pallas import tpu as pltpu
from jax.experimental.pallas import tpu_sc as plsc

EMBED_DIM = 64
SEQ = 200
LANES = 16

NUM_CORES = 2
NUM_SUBCORES = 16
NUM_WORKERS = NUM_CORES * NUM_SUBCORES

BBLK = 128
NBUF = 2
NST = 2
GROUP = 2


def _pe_table():
    pos = jnp.arange(SEQ, dtype=jnp.float32)[:, None]
    i = jnp.arange(0, EMBED_DIM, 2, dtype=jnp.float32)
    div = jnp.exp(-(jnp.log(10000.0) * i / EMBED_DIM))
    pe = jnp.zeros((SEQ, EMBED_DIM), dtype=jnp.float32)
    pe = pe.at[:, 0::2].set(jnp.sin(pos * div))
    pe = pe.at[:, 1::2].set(jnp.cos(pos * div))
    return pe


def _make_kernel(batch, seq):
    assert batch == NUM_WORKERS * BBLK and seq == SEQ
    steps = seq
    mesh = plsc.VectorSubcoreMesh(
        core_axis_name="c", subcore_axis_name="s",
        num_cores=NUM_CORES, num_subcores=NUM_SUBCORES)

    @functools.partial(
        pl.kernel,
        out_type=jax.ShapeDtypeStruct(
            (seq, EMBED_DIM // 8, NUM_WORKERS, 8, BBLK), jnp.float32),
        mesh=mesh,
        compiler_params=pltpu.CompilerParams(
            use_tc_tiling_on_sc=True, needs_layout_passes=False),
        scratch_types=[
            pltpu.VMEM((steps, BBLK), jnp.int32),
            pltpu.VMEM((SEQ, EMBED_DIM), jnp.float32),
            pltpu.VMEM((NBUF, BBLK), jnp.int32),
            pltpu.VMEM((BBLK,), jnp.int32),
            pltpu.VMEM((NBUF, BBLK, 128), jnp.float32),
            pltpu.VMEM((NST, 8, 8, BBLK), jnp.float32),
        ]
        + [pltpu.SemaphoreType.DMA] * (NBUF + NST),
    )
    def k(x_hbm, tbl_hbm, pe_hbm, out_hbm, idx_v, pe_v, p_v, h_v, gath_v,
          stg_v, *sems):
        sem_g = sems[:NBUF]
        sem_s = sems[NBUF:]
        wid = lax.axis_index("s") * NUM_CORES + lax.axis_index("c")
        pltpu.sync_copy(x_hbm.at[wid], idx_v)
        pltpu.sync_copy(pe_hbm, pe_v)

        iota = lax.iota(jnp.int32, LANES)

        def launch_gather(b, s):
            for g in range(BBLK // LANES):
                sl = pl.ds(g * LANES, LANES)
                p_v[b, sl] = lax.shift_right_logical(idx_v[s, sl], 1)
            pltpu.async_copy(tbl_hbm.at[p_v.at[b]], gath_v.at[b], sem_g[b])

        def wait_gather(b):
            pltpu.make_async_copy(
                tbl_hbm.at[pl.ds(0, BBLK)], gath_v.at[b], sem_g[b]).wait()

        def launch_scatter(t, s):
            pltpu.async_copy(stg_v.at[t], out_hbm.at[s, :, wid], sem_s[t])

        def wait_scatter(t):
            pltpu.make_async_copy(
                stg_v.at[t], out_hbm.at[0, :, 0], sem_s[t]).wait()

        def process(b, t, s):
            for g in range(BBLK // LANES):
                sl = pl.ds(g * LANES, LANES)
                h_v[sl] = lax.shift_left(
                    lax.bitwise_and(idx_v[s, sl], 1), 6)

            def c_body(c, _):
                pe16 = pe_v[s, pl.ds(c * LANES, LANES)]

                def g_body(g, _):
                    half_g = h_v[pl.ds(g * LANES, LANES)]
                    rows_g = iota + g * LANES
                    gsl = pl.ds(g * LANES, LANES)
                    for sd16 in range(LANES):
                        d = c * LANES + sd16
                        jd = 2 * c + sd16 // 8
                        sd = sd16 % 8
                        vals = plsc.load_gather(
                            gath_v.at[b], [rows_g, half_g + d])
                        stg_v[t, jd, sd, gsl] = vals + pe16[sd16]
                    return 0

                lax.fori_loop(0, BBLK // LANES, g_body, 0)
                return 0

            lax.fori_loop(0, EMBED_DIM // LANES, c_body, 0)

        launch_gather(0, 0)
        launch_gather(1, 1)

        def step_body(s, b, t, *, first, last):
            wait_gather(b)
            if not first:
                wait_scatter(t)
            process(b, t, s)
            launch_scatter(t, s)
            if not last:
                launch_gather((b + 2) % NBUF, s + 2)

        for ss in range(GROUP):
            step_body(ss, ss % NBUF, ss % NST, first=ss < NST, last=False)

        groups = steps // GROUP - 2

        def group_body(gi, _):
            s0 = GROUP + gi * GROUP
            for off in range(GROUP):
                step_body(s0 + off, off % NBUF, off % NST,
                          first=False, last=False)
            return 0

        lax.fori_loop(0, groups, group_body, 0)

        for off in range(GROUP):
            ss = steps - GROUP + off
            step_body(ss, ss % NBUF, ss % NST, first=False,
                      last=ss + 2 >= steps)

        for t in range(NST):
            wait_scatter(t)

    return k


def kernel(x, token_embedding_weight):
    batch, seq = x.shape
    xi = (x.astype(jnp.int32).T.reshape(seq, NUM_WORKERS, BBLK)
          .transpose(1, 0, 2))
    tbl = token_embedding_weight.reshape(500000, 128)
    pe = _pe_table()
    k = _make_kernel(batch, seq)
    out5 = k(xi, tbl, pe)
    return out5.transpose(2, 4, 0, 1, 3).reshape(batch, seq, EMBED_DIM)
